# Initial kernel scaffold; baseline (speedup 1.0000x reference)
import jax
import jax.numpy as jnp
from jax.experimental import pallas as pl


def kernel(target, context, negative_samples, embeddings, context_embeddings):
    raise NotImplementedError("write your pallas kernel here")


# speedup vs baseline: 2.0181x; 2.0181x over previous
import functools

import jax
import jax.numpy as jnp
from jax.experimental import pallas as pl
from jax.experimental.pallas import tpu as pltpu
from jax.experimental.pallas import tpu_sc as plsc

DIM = 128
BATCH = 16384
NEG = 20

_WINDOW = 128
_BB = 512
_NBLK = BATCH // _BB
_KBLK = 1 + NEG


def _sc_gather(table, idx, n_idx):
    mesh = plsc.VectorSubcoreMesh(core_axis_name="c", subcore_axis_name="s")

    @functools.partial(
        pl.kernel,
        out_type=jax.ShapeDtypeStruct((n_idx, DIM), jnp.float32),
        mesh=mesh,
    )
    def gather_kernel(tab_hbm, idx_hbm, out_hbm):
        def body(i_vmem, o_vmem):
            pltpu.sync_copy(tab_hbm.at[i_vmem.at[0]], o_vmem)

        pltpu.emit_pipeline(
            body,
            grid=(n_idx // _WINDOW,),
            in_specs=[pl.BlockSpec((1, _WINDOW), lambda i: (0, i))],
            out_specs=[pl.BlockSpec((_WINDOW, DIM), lambda i: (i, 0))],
            core_axis_name=("c", "s"),
            dimension_semantics=(pltpu.PARALLEL,),
        )(idx_hbm, out_hbm)

    return gather_kernel(table, idx.reshape(1, n_idx))


def _loss_body(t_ref, r_ref, out_ref):
    i = pl.program_id(0)
    k = pl.program_id(1)

    @pl.when(jnp.logical_and(i == 0, k == 0))
    def _():
        out_ref[...] = jnp.zeros((1, 1), jnp.float32)

    t = t_ref[...]
    d = jnp.sum(t * r_ref[...], axis=1, keepdims=True)
    s = jnp.where(k == 0, jax.nn.log_sigmoid(d), jax.nn.log_sigmoid(-d))
    out_ref[...] += jnp.sum(s).reshape(1, 1)

    @pl.when(jnp.logical_and(i == _NBLK - 1, k == _KBLK - 1))
    def _():
        out_ref[...] *= -1.0 / BATCH


def kernel(target, context, negative_samples, embeddings, context_embeddings):
    tgt = target.astype(jnp.int32)
    ctx = context.astype(jnp.int32)
    neg = negative_samples.astype(jnp.int32)

    cn_idx = jnp.concatenate([ctx, neg.T.reshape(-1)])
    t_rows = _sc_gather(embeddings, tgt, BATCH)
    cn_rows = _sc_gather(context_embeddings, cn_idx, _KBLK * BATCH)

    loss = pl.pallas_call(
        _loss_body,
        grid=(_NBLK, _KBLK),
        in_specs=[
            pl.BlockSpec((_BB, DIM), lambda i, k: (i, 0)),
            pl.BlockSpec((_BB, DIM), lambda i, k: (k * _NBLK + i, 0)),
        ],
        out_specs=pl.BlockSpec((1, 1), lambda i, k: (0, 0)),
        out_shape=jax.ShapeDtypeStruct((1, 1), jnp.float32),
    )(t_rows, cn_rows)
    return loss[0, 0]


# speedup vs baseline: 3.2696x; 1.6202x over previous
import functools

import jax
import jax.numpy as jnp
from jax.experimental import pallas as pl
from jax.experimental.pallas import tpu as pltpu
from jax.experimental.pallas import tpu_sc as plsc

DIM = 128
BATCH = 16384
NEG = 20

_WINDOW = 128
_BB = 512
_NBLK = BATCH // _BB
_KBLK = 1 + NEG


def _sc_gather(table, idx, n_idx):
    mesh = plsc.VectorSubcoreMesh(core_axis_name="c", subcore_axis_name="s")

    @functools.partial(
        pl.kernel,
        out_type=jax.ShapeDtypeStruct((n_idx, DIM), jnp.float32),
        mesh=mesh,
    )
    def gather_kernel(tab_hbm, idx_hbm, out_hbm):
        def body(i_vmem, o_vmem):
            pltpu.sync_copy(tab_hbm.at[i_vmem.at[0]], o_vmem)

        pltpu.emit_pipeline(
            body,
            grid=(n_idx // _WINDOW,),
            in_specs=[pl.BlockSpec((1, _WINDOW), lambda i: (0, i))],
            out_specs=[pl.BlockSpec((_WINDOW, DIM), lambda i: (i, 0))],
            core_axis_name=("c", "s"),
            dimension_semantics=(pltpu.PARALLEL,),
        )(idx_hbm, out_hbm)

    return gather_kernel(table, idx.reshape(1, n_idx))


def _loss_body(t_ref, r_ref, out_ref):
    k = pl.program_id(0)

    @pl.when(k == 0)
    def _():
        out_ref[...] = jnp.zeros((1, 1), jnp.float32)

    t = t_ref[...]
    d = jnp.sum(t * r_ref[...], axis=1, keepdims=True)
    s = jnp.where(k == 0, jax.nn.log_sigmoid(d), jax.nn.log_sigmoid(-d))
    out_ref[...] += jnp.sum(s).reshape(1, 1)

    @pl.when(k == _KBLK - 1)
    def _():
        out_ref[...] *= -1.0 / BATCH


def kernel(target, context, negative_samples, embeddings, context_embeddings):
    tgt = target.astype(jnp.int32)
    ctx = context.astype(jnp.int32)
    neg = negative_samples.astype(jnp.int32)

    cn_idx = jnp.concatenate([ctx, neg.T.reshape(-1)])
    t_rows = _sc_gather(embeddings, tgt, BATCH)
    cn_rows = _sc_gather(context_embeddings, cn_idx, _KBLK * BATCH)

    loss = pl.pallas_call(
        _loss_body,
        grid=(_KBLK,),
        in_specs=[
            pl.BlockSpec((BATCH, DIM), lambda k: (0, 0)),
            pl.BlockSpec((BATCH, DIM), lambda k: (k, 0)),
        ],
        out_specs=pl.BlockSpec((1, 1), lambda k: (0, 0)),
        out_shape=jax.ShapeDtypeStruct((1, 1), jnp.float32),
    )(t_rows, cn_rows)
    return loss[0, 0]


# speedup vs baseline: 3.9953x; 1.2220x over previous
import functools

import jax
import jax.numpy as jnp
from jax.experimental import pallas as pl
from jax.experimental.pallas import tpu as pltpu
from jax.experimental.pallas import tpu_sc as plsc

DIM = 128
BATCH = 16384
NEG = 20

_WINDOW = 128
_KBLK = 1 + NEG


def _sc_gather_all(embeddings, context_embeddings, tgt_idx, cn_idx):
    mesh = plsc.VectorSubcoreMesh(core_axis_name="c", subcore_axis_name="s")

    @functools.partial(
        pl.kernel,
        out_type=(
            jax.ShapeDtypeStruct((BATCH, DIM), jnp.float32),
            jax.ShapeDtypeStruct((_KBLK * BATCH, DIM), jnp.float32),
        ),
        mesh=mesh,
    )
    def gather_kernel(emb_hbm, cemb_hbm, ti_hbm, cni_hbm, t_out, cn_out):
        def body_t(i_vmem, o_vmem):
            pltpu.sync_copy(emb_hbm.at[i_vmem.at[0]], o_vmem)

        pltpu.emit_pipeline(
            body_t,
            grid=(BATCH // _WINDOW,),
            in_specs=[pl.BlockSpec((1, _WINDOW), lambda i: (0, i))],
            out_specs=[pl.BlockSpec((_WINDOW, DIM), lambda i: (i, 0))],
            core_axis_name=("c", "s"),
            dimension_semantics=(pltpu.PARALLEL,),
        )(ti_hbm, t_out)

        def body_cn(i_vmem, o_vmem):
            pltpu.sync_copy(cemb_hbm.at[i_vmem.at[0]], o_vmem)

        pltpu.emit_pipeline(
            body_cn,
            grid=(_KBLK * BATCH // _WINDOW,),
            in_specs=[pl.BlockSpec((1, _WINDOW), lambda i: (0, i))],
            out_specs=[pl.BlockSpec((_WINDOW, DIM), lambda i: (i, 0))],
            core_axis_name=("c", "s"),
            dimension_semantics=(pltpu.PARALLEL,),
        )(cni_hbm, cn_out)

    return gather_kernel(
        embeddings,
        context_embeddings,
        tgt_idx.reshape(1, BATCH),
        cn_idx.reshape(1, _KBLK * BATCH),
    )


def _dots_body(t_ref, r_ref, out_ref):
    d = jnp.sum(t_ref[...] * r_ref[...], axis=1, keepdims=True)
    out_ref[...] = d.reshape(1, BATCH, 1)


def _finish_body(d_ref, out_ref):
    dm = d_ref[...]
    row = jax.lax.broadcasted_iota(jnp.int32, (_KBLK, BATCH), 0)
    s = jnp.where(row == 0, jax.nn.log_sigmoid(dm), jax.nn.log_sigmoid(-dm))
    out_ref[...] = (jnp.sum(s) * (-1.0 / BATCH)).reshape(1, 1)


def kernel(target, context, negative_samples, embeddings, context_embeddings):
    tgt = target.astype(jnp.int32)
    ctx = context.astype(jnp.int32)
    neg = negative_samples.astype(jnp.int32)

    cn_idx = jnp.concatenate([ctx, neg.T.reshape(-1)])
    t_rows, cn_rows = _sc_gather_all(embeddings, context_embeddings, tgt, cn_idx)

    dots = pl.pallas_call(
        _dots_body,
        grid=(_KBLK,),
        in_specs=[
            pl.BlockSpec((BATCH, DIM), lambda k: (0, 0)),
            pl.BlockSpec((BATCH, DIM), lambda k: (k, 0)),
        ],
        out_specs=pl.BlockSpec((1, BATCH, 1), lambda k: (k, 0, 0)),
        out_shape=jax.ShapeDtypeStruct((_KBLK, BATCH, 1), jnp.float32),
    )(t_rows, cn_rows)

    loss = pl.pallas_call(
        _finish_body,
        in_specs=[pl.BlockSpec((_KBLK, BATCH), lambda: (0, 0))],
        out_specs=pl.BlockSpec((1, 1), lambda: (0, 0)),
        out_shape=jax.ShapeDtypeStruct((1, 1), jnp.float32),
    )(dots.reshape(_KBLK, BATCH))
    return loss[0, 0]


# speedup vs baseline: 5.6093x; 1.4040x over previous
import functools

import jax
import jax.numpy as jnp
from jax.experimental import pallas as pl
from jax.experimental.pallas import tpu as pltpu
from jax.experimental.pallas import tpu_sc as plsc

DIM = 128
BATCH = 16384
NEG = 20

_WINDOW = 128
_KBLK = 1 + NEG


def _sc_gather_all(embeddings, context_embeddings, tgt_idx, cn_idx):
    mesh = plsc.VectorSubcoreMesh(core_axis_name="c", subcore_axis_name="s")

    @functools.partial(
        pl.kernel,
        out_type=(
            jax.ShapeDtypeStruct((BATCH, DIM), jnp.float32),
            jax.ShapeDtypeStruct((_KBLK * BATCH, DIM), jnp.float32),
        ),
        mesh=mesh,
    )
    def gather_kernel(emb_hbm, cemb_hbm, ti_hbm, cni_hbm, t_out, cn_out):
        def body_t(i_vmem, o_vmem):
            pltpu.sync_copy(emb_hbm.at[i_vmem.at[0]], o_vmem)

        pltpu.emit_pipeline(
            body_t,
            grid=(BATCH // _WINDOW,),
            in_specs=[pl.BlockSpec((1, _WINDOW), lambda i: (0, i))],
            out_specs=[pl.BlockSpec((_WINDOW, DIM), lambda i: (i, 0))],
            core_axis_name=("c", "s"),
            dimension_semantics=(pltpu.PARALLEL,),
        )(ti_hbm, t_out)

        def body_cn(i_vmem, o_vmem):
            pltpu.sync_copy(cemb_hbm.at[i_vmem.at[0]], o_vmem)

        pltpu.emit_pipeline(
            body_cn,
            grid=(_KBLK * BATCH // _WINDOW,),
            in_specs=[pl.BlockSpec((1, _WINDOW), lambda i: (0, i))],
            out_specs=[pl.BlockSpec((_WINDOW, DIM), lambda i: (i, 0))],
            core_axis_name=("c", "s"),
            dimension_semantics=(pltpu.PARALLEL,),
        )(cni_hbm, cn_out)

    return gather_kernel(
        embeddings,
        context_embeddings,
        tgt_idx.reshape(1, BATCH),
        cn_idx.reshape(1, _KBLK * BATCH),
    )


def _dots_body(t_ref, r_ref, out_ref):
    p = t_ref[...] * r_ref[...]
    d = jnp.sum(p.T, axis=0, keepdims=True)
    out_ref[...] = d.reshape(1, 1, BATCH)


def _finish_body(d_ref, out_ref):
    dm = d_ref[...]
    row = jax.lax.broadcasted_iota(jnp.int32, (_KBLK, 1, BATCH), 0)
    s = jnp.where(row == 0, jax.nn.log_sigmoid(dm), jax.nn.log_sigmoid(-dm))
    out_ref[...] = (jnp.sum(s) * (-1.0 / BATCH)).reshape(1, 1)


def kernel(target, context, negative_samples, embeddings, context_embeddings):
    tgt = target.astype(jnp.int32)
    ctx = context.astype(jnp.int32)
    neg = negative_samples.astype(jnp.int32)

    cn_idx = jnp.concatenate([ctx, neg.T.reshape(-1)])
    t_rows, cn_rows = _sc_gather_all(embeddings, context_embeddings, tgt, cn_idx)

    dots = pl.pallas_call(
        _dots_body,
        grid=(_KBLK,),
        in_specs=[
            pl.BlockSpec((BATCH, DIM), lambda k: (0, 0)),
            pl.BlockSpec((BATCH, DIM), lambda k: (k, 0)),
        ],
        out_specs=pl.BlockSpec((1, 1, BATCH), lambda k: (k, 0, 0)),
        out_shape=jax.ShapeDtypeStruct((_KBLK, 1, BATCH), jnp.float32),
    )(t_rows, cn_rows)

    loss = pl.pallas_call(
        _finish_body,
        in_specs=[pl.BlockSpec((_KBLK, 1, BATCH), lambda: (0, 0, 0))],
        out_specs=pl.BlockSpec((1, 1), lambda: (0, 0)),
        out_shape=jax.ShapeDtypeStruct((1, 1), jnp.float32),
    )(dots)
    return loss[0, 0]


# speedup vs baseline: 6.4499x; 1.1498x over previous
import functools

import jax
import jax.numpy as jnp
from jax.experimental import pallas as pl
from jax.experimental.pallas import tpu as pltpu
from jax.experimental.pallas import tpu_sc as plsc

DIM = 128
BATCH = 16384
NEG = 20

_WINDOW = 128
_KBLK = 1 + NEG


_NW = 32
_TW = BATCH // (_NW * _WINDOW)
_CNW = _KBLK * BATCH // (_NW * _WINDOW)
_NBUF = 6
_GRP = _NBUF // 2


def _sc_gather_all(embeddings, context_embeddings, tgt_idx, cn_idx):
    mesh = plsc.VectorSubcoreMesh(core_axis_name="c", subcore_axis_name="s")

    @functools.partial(
        pl.kernel,
        out_type=(
            jax.ShapeDtypeStruct((BATCH, DIM), jnp.float32),
            jax.ShapeDtypeStruct((_KBLK * BATCH, DIM), jnp.float32),
        ),
        mesh=mesh,
        scratch_types=(
            [pltpu.VMEM((_TW * _WINDOW,), jnp.int32),
             pltpu.VMEM((_CNW * _WINDOW,), jnp.int32)]
            + [pltpu.VMEM((_WINDOW, DIM), jnp.float32)] * _NBUF
            + [pltpu.SemaphoreType.DMA] * (2 * _NBUF)
        ),
    )
    def gather_kernel(emb_hbm, cemb_hbm, ti_hbm, cni_hbm, t_out, cn_out,
                      ti_v, cni_v, *scr):
        bufs = scr[:_NBUF]
        gsem = scr[_NBUF:2 * _NBUF]
        wsem = scr[2 * _NBUF:]
        wid = jax.lax.axis_index("s") * 2 + jax.lax.axis_index("c")
        t_base = wid * (_TW * _WINDOW)
        cn_base = wid * (_CNW * _WINDOW)

        pltpu.sync_copy(ti_hbm.at[pl.ds(t_base, _TW * _WINDOW)], ti_v)
        pltpu.sync_copy(cni_hbm.at[pl.ds(cn_base, _CNW * _WINDOW)], cni_v)

        th = []
        for w in range(_TW):
            th.append(pltpu.async_copy(
                emb_hbm.at[ti_v.at[pl.ds(w * _WINDOW, _WINDOW)]],
                bufs[w], gsem[w]))
        tw = []
        for w in range(_TW):
            th[w].wait()
            tw.append(pltpu.async_copy(
                bufs[w], t_out.at[pl.ds(t_base + w * _WINDOW, _WINDOW)],
                wsem[w]))
        for w in range(_TW):
            tw[w].wait()

        def cn_gather(w, b):
            return pltpu.async_copy(
                cemb_hbm.at[cni_v.at[pl.ds(w * _WINDOW, _WINDOW)]],
                bufs[b], gsem[b])

        def cn_write(w, b):
            return pltpu.async_copy(
                bufs[b], cn_out.at[pl.ds(cn_base + w * _WINDOW, _WINDOW)],
                wsem[b])

        def wait_write(b):
            pltpu.make_async_copy(
                bufs[b], cn_out.at[pl.ds(cn_base, _WINDOW)], wsem[b]).wait()

        def group(w0, first):
            for g in range(2):
                bb = g * _GRP
                if not first:
                    for b in range(_GRP):
                        wait_write(bb + b)
                gh = [cn_gather(w0 + g * _GRP + b, bb + b)
                      for b in range(_GRP)]
                for b in range(_GRP):
                    gh[b].wait()
                    cn_write(w0 + g * _GRP + b, bb + b)

        group(0, True)

        @pl.loop(_NBUF, _CNW, step=_NBUF)
        def _(w0):
            group(w0, False)

        for b in range(_NBUF):
            wait_write(b)

    return gather_kernel(
        embeddings, context_embeddings, tgt_idx, cn_idx)


def _dots_body(t_ref, r_ref, out_ref):
    p = t_ref[...] * r_ref[...]
    d = jnp.sum(p.T, axis=0, keepdims=True)
    out_ref[...] = d.reshape(1, 1, BATCH)


def _finish_body(d_ref, out_ref):
    dm = d_ref[...]
    row = jax.lax.broadcasted_iota(jnp.int32, (_KBLK, 1, BATCH), 0)
    s = jnp.where(row == 0, jax.nn.log_sigmoid(dm), jax.nn.log_sigmoid(-dm))
    out_ref[...] = (jnp.sum(s) * (-1.0 / BATCH)).reshape(1, 1)


def kernel(target, context, negative_samples, embeddings, context_embeddings):
    tgt = target.astype(jnp.int32)
    ctx = context.astype(jnp.int32)
    neg = negative_samples.astype(jnp.int32)

    cn_idx = jnp.concatenate([ctx, neg.T.reshape(-1)])
    t_rows, cn_rows = _sc_gather_all(embeddings, context_embeddings, tgt, cn_idx)

    dots = pl.pallas_call(
        _dots_body,
        grid=(_KBLK,),
        in_specs=[
            pl.BlockSpec((BATCH, DIM), lambda k: (0, 0)),
            pl.BlockSpec((BATCH, DIM), lambda k: (k, 0)),
        ],
        out_specs=pl.BlockSpec((1, 1, BATCH), lambda k: (k, 0, 0)),
        out_shape=jax.ShapeDtypeStruct((_KBLK, 1, BATCH), jnp.float32),
    )(t_rows, cn_rows)

    loss = pl.pallas_call(
        _finish_body,
        in_specs=[pl.BlockSpec((_KBLK, 1, BATCH), lambda: (0, 0, 0))],
        out_specs=pl.BlockSpec((1, 1), lambda: (0, 0)),
        out_shape=jax.ShapeDtypeStruct((1, 1), jnp.float32),
    )(dots)
    return loss[0, 0]


# speedup vs baseline: 10.4587x; 1.6215x over previous
import dataclasses
import functools

import jax
import jax.numpy as jnp
from jax.experimental import pallas as pl
from jax.experimental.pallas import tpu as pltpu
from jax.experimental.pallas import tpu_sc as plsc

DIM = 128
BATCH = 16384
NEG = 20
_KBLK = 1 + NEG
_KPAD = 32

_NW = 32
_EPB = BATCH // _NW
_EW = 8
_ROWS = _EW * _KBLK
_WIN = _EPB // _EW
_NCH = DIM // 16


def _sc_dots(embeddings, context_embeddings, tgt_idx, cn_idx):
    mesh = plsc.VectorSubcoreMesh(core_axis_name="c", subcore_axis_name="s")
    cp = pltpu.CompilerParams()
    if "needs_layout_passes" in pltpu.CompilerParams.__dataclass_fields__:
        cp = dataclasses.replace(cp, needs_layout_passes=False)

    @functools.partial(
        pl.kernel,
        out_type=jax.ShapeDtypeStruct((BATCH, _KPAD), jnp.float32),
        mesh=mesh,
        compiler_params=cp,
        scratch_types=(
            [pltpu.VMEM((_EPB,), jnp.int32),
             pltpu.VMEM((_EPB * _KBLK,), jnp.int32),
             pltpu.VMEM((_EPB, _KPAD), jnp.float32)]
            + [pltpu.VMEM((_ROWS, DIM), jnp.float32)] * 2
            + [pltpu.VMEM((_EW, DIM), jnp.float32)] * 2
            + [pltpu.SemaphoreType.DMA] * 2
        ),
    )
    def dots_kernel(emb_hbm, cemb_hbm, ti_hbm, cni_hbm, out_hbm,
                    ti_v, cni_v, dots_v, cn0, cn1, tb0, tb1, sem0, sem1):
        cnb = (cn0, cn1)
        tbb = (tb0, tb1)
        sems = (sem0, sem1)
        wid = jax.lax.axis_index("s") * 2 + jax.lax.axis_index("c")
        el_base = wid * _EPB

        pltpu.sync_copy(ti_hbm.at[pl.ds(el_base, _EPB)], ti_v)
        pltpu.sync_copy(cni_hbm.at[pl.ds(el_base * _KBLK, _EPB * _KBLK)],
                        cni_v)

        def start_gather(w, b):
            off = w * _ROWS
            h = [
                pltpu.async_copy(
                    cemb_hbm.at[cni_v.at[pl.ds(off, 128)]],
                    cnb[b].at[pl.ds(0, 128)], sems[b]),
                pltpu.async_copy(
                    cemb_hbm.at[cni_v.at[pl.ds(off + 128, _ROWS - 128)]],
                    cnb[b].at[pl.ds(128, _ROWS - 128)], sems[b]),
                pltpu.async_copy(
                    emb_hbm.at[ti_v.at[pl.ds(w * _EW, _EW)]],
                    tbb[b], sems[b]),
            ]
            return h

        def wait_gather(b):
            pltpu.make_async_copy(
                cemb_hbm.at[cni_v.at[pl.ds(0, 128)]],
                cnb[b].at[pl.ds(0, 128)], sems[b]).wait()
            pltpu.make_async_copy(
                cemb_hbm.at[cni_v.at[pl.ds(0, _ROWS - 128)]],
                cnb[b].at[pl.ds(128, _ROWS - 128)], sems[b]).wait()
            pltpu.make_async_copy(
                emb_hbm.at[ti_v.at[pl.ds(0, _EW)]], tbb[b], sems[b]).wait()

        lanes = jax.lax.iota(jnp.int32, 16)

        def compute(w, b):
            @pl.loop(0, _EW)
            def _(e):
                t = [tbb[b][e, pl.ds(j * 16, 16)] for j in range(_NCH)]
                col = w * _EW + e
                v = [jnp.zeros((16,), jnp.float32) for _ in range(2)]
                for k in range(_KBLK):
                    row = e * _KBLK + k
                    acc = t[0] * cnb[b][row, pl.ds(0, 16)]
                    for j in range(1, _NCH):
                        acc = acc + t[j] * cnb[b][row, pl.ds(j * 16, 16)]
                    s = jnp.broadcast_to(jnp.sum(acc), (16,))
                    h = k // 16
                    v[h] = jnp.where(lanes == (k % 16), s, v[h])
                dots_v[col, pl.ds(0, 16)] = v[0]
                dots_v[col, pl.ds(16, 16)] = v[1]

        start_gather(0, 0)
        start_gather(1, 1)

        @pl.loop(0, _WIN - 2, step=2)
        def _(w0):
            wait_gather(0)
            compute(w0, 0)
            start_gather(w0 + 2, 0)
            wait_gather(1)
            compute(w0 + 1, 1)
            start_gather(w0 + 3, 1)

        wait_gather(0)
        compute(_WIN - 2, 0)
        wait_gather(1)
        compute(_WIN - 1, 1)

        pltpu.sync_copy(dots_v, out_hbm.at[pl.ds(el_base, _EPB)])

    return dots_kernel(embeddings, context_embeddings, tgt_idx, cn_idx)


_FIN_R = BATCH * _KPAD // DIM


def _finish_body(d_ref, out_ref):
    dm = d_ref[...]
    col = jax.lax.broadcasted_iota(jnp.int32, (_FIN_R, DIM), 1) % _KPAD
    s = jnp.where(col == 0, jax.nn.log_sigmoid(dm), jax.nn.log_sigmoid(-dm))
    s = jnp.where(col < _KBLK, s, 0.0)
    out_ref[...] = (jnp.sum(s) * (-1.0 / BATCH)).reshape(1, 1)


def kernel(target, context, negative_samples, embeddings, context_embeddings):
    tgt = target.astype(jnp.int32)
    ctx = context.astype(jnp.int32)
    neg = negative_samples.astype(jnp.int32)

    cn_idx = jnp.concatenate([ctx[:, None], neg], axis=1).reshape(-1)

    dots = _sc_dots(embeddings, context_embeddings, tgt, cn_idx)

    loss = pl.pallas_call(
        _finish_body,
        in_specs=[pl.BlockSpec((_FIN_R, DIM), lambda: (0, 0))],
        out_specs=pl.BlockSpec((1, 1), lambda: (0, 0)),
        out_shape=jax.ShapeDtypeStruct((1, 1), jnp.float32),
    )(dots.reshape(_FIN_R, DIM))
    return loss[0, 0]


# speedup vs baseline: 11.1578x; 1.0668x over previous
import dataclasses
import functools

import jax
import jax.numpy as jnp
from jax.experimental import pallas as pl
from jax.experimental.pallas import tpu as pltpu
from jax.experimental.pallas import tpu_sc as plsc

DIM = 128
BATCH = 16384
NEG = 20
_KBLK = 1 + NEG
_KPAD = 32

_NW = 32
_EPB = BATCH // _NW
_EW = 8
_ROWS = _EW * _KBLK
_WIN = _EPB // _EW
_NCH = DIM // 16


def _sc_dots(embeddings, context_embeddings, tgt_idx, cn_idx):
    mesh = plsc.VectorSubcoreMesh(core_axis_name="c", subcore_axis_name="s")
    cp = pltpu.CompilerParams()
    if "needs_layout_passes" in pltpu.CompilerParams.__dataclass_fields__:
        cp = dataclasses.replace(cp, needs_layout_passes=False)

    @functools.partial(
        pl.kernel,
        out_type=jax.ShapeDtypeStruct((BATCH * _KPAD,), jnp.float32),
        mesh=mesh,
        compiler_params=cp,
        scratch_types=(
            [pltpu.VMEM((_EPB,), jnp.int32),
             pltpu.VMEM((_EPB * _KBLK,), jnp.int32),
             pltpu.VMEM((_EPB * _KPAD,), jnp.float32)]
            + [pltpu.VMEM((_ROWS, DIM), jnp.float32)] * 2
            + [pltpu.VMEM((_EW, DIM), jnp.float32)] * 2
            + [pltpu.SemaphoreType.DMA] * 2
        ),
    )
    def dots_kernel(emb_hbm, cemb_hbm, ti_hbm, cni_hbm, out_hbm,
                    ti_v, cni_v, dots_v, cn0, cn1, tb0, tb1, sem0, sem1):
        cnb = (cn0, cn1)
        tbb = (tb0, tb1)
        sems = (sem0, sem1)
        wid = jax.lax.axis_index("s") * 2 + jax.lax.axis_index("c")
        el_base = wid * _EPB

        pltpu.sync_copy(ti_hbm.at[pl.ds(el_base, _EPB)], ti_v)
        pltpu.sync_copy(cni_hbm.at[pl.ds(el_base * _KBLK, _EPB * _KBLK)],
                        cni_v)

        def start_gather(w, b):
            off = w * _ROWS
            h = [
                pltpu.async_copy(
                    cemb_hbm.at[cni_v.at[pl.ds(off, 128)]],
                    cnb[b].at[pl.ds(0, 128)], sems[b]),
                pltpu.async_copy(
                    cemb_hbm.at[cni_v.at[pl.ds(off + 128, _ROWS - 128)]],
                    cnb[b].at[pl.ds(128, _ROWS - 128)], sems[b]),
                pltpu.async_copy(
                    emb_hbm.at[ti_v.at[pl.ds(w * _EW, _EW)]],
                    tbb[b], sems[b]),
            ]
            return h

        def wait_gather(b):
            pltpu.make_async_copy(
                cemb_hbm.at[cni_v.at[pl.ds(0, 128)]],
                cnb[b].at[pl.ds(0, 128)], sems[b]).wait()
            pltpu.make_async_copy(
                cemb_hbm.at[cni_v.at[pl.ds(0, _ROWS - 128)]],
                cnb[b].at[pl.ds(128, _ROWS - 128)], sems[b]).wait()
            pltpu.make_async_copy(
                emb_hbm.at[ti_v.at[pl.ds(0, _EW)]], tbb[b], sems[b]).wait()

        lanes = jax.lax.iota(jnp.int32, 16)

        def compute(w, b):
            @pl.loop(0, _EW, step=2)
            def _(e0):
                for de in range(2):
                    e = e0 + de
                    t = [tbb[b][e, pl.ds(j * 16, 16)] for j in range(_NCH)]
                    col = w * _EW + e
                    v = [jnp.zeros((16,), jnp.float32) for _ in range(2)]
                    for k in range(_KBLK):
                        row = e * _KBLK + k
                        acc = t[0] * cnb[b][row, pl.ds(0, 16)]
                        for j in range(1, _NCH):
                            acc = acc + t[j] * cnb[b][row, pl.ds(j * 16, 16)]
                        s = jnp.broadcast_to(jnp.sum(acc), (16,))
                        h = k // 16
                        v[h] = jnp.where(lanes == (k % 16), s, v[h])
                    dots_v[pl.ds(col * _KPAD, 16)] = v[0]
                    dots_v[pl.ds(col * _KPAD + 16, 16)] = v[1]

        start_gather(0, 0)
        start_gather(1, 1)

        @pl.loop(0, _WIN - 2, step=2)
        def _(w0):
            wait_gather(0)
            compute(w0, 0)
            start_gather(w0 + 2, 0)
            wait_gather(1)
            compute(w0 + 1, 1)
            start_gather(w0 + 3, 1)

        wait_gather(0)
        compute(_WIN - 2, 0)
        wait_gather(1)
        compute(_WIN - 1, 1)

        pltpu.sync_copy(dots_v,
                        out_hbm.at[pl.ds(el_base * _KPAD, _EPB * _KPAD)])

    return dots_kernel(embeddings, context_embeddings, tgt_idx, cn_idx)


_FIN_R = BATCH * _KPAD // DIM


def _finish_body(d_ref, out_ref):
    dm = d_ref[...]
    col = jax.lax.broadcasted_iota(jnp.int32, (_FIN_R, DIM), 1) % _KPAD
    s = jnp.where(col == 0, jax.nn.log_sigmoid(dm), jax.nn.log_sigmoid(-dm))
    s = jnp.where(col < _KBLK, s, 0.0)
    out_ref[...] = (jnp.sum(s) * (-1.0 / BATCH)).reshape(1, 1)


def kernel(target, context, negative_samples, embeddings, context_embeddings):
    tgt = target.astype(jnp.int32)
    ctx = context.astype(jnp.int32)
    neg = negative_samples.astype(jnp.int32)

    cn_idx = jnp.concatenate([ctx[:, None], neg], axis=1).reshape(-1)

    dots = _sc_dots(embeddings, context_embeddings, tgt, cn_idx)

    loss = pl.pallas_call(
        _finish_body,
        in_specs=[pl.BlockSpec((_FIN_R, DIM), lambda: (0, 0))],
        out_specs=pl.BlockSpec((1, 1), lambda: (0, 0)),
        out_shape=jax.ShapeDtypeStruct((1, 1), jnp.float32),
    )(dots.reshape(_FIN_R, DIM))
    return loss[0, 0]
